# baseline (speedup 1.0000x reference)
import jax
import jax.numpy as jnp
from jax.experimental import pallas as pl
from jax.experimental.pallas import tpu as pltpu
from jax.experimental.pallas import tpu_sc as plsc

_B = 4096
_D_IN = 128
_D_H = 256
_D_OUT = 64
_N_TYPES = 256
_MESA = 8
_S1 = _D_IN * _D_H
_S2 = _S1 + _D_H
_S3 = _S2 + _D_H * _D_OUT
_K = _MESA + 1
_KP = 128

_BLK = 512
_GW = 128


def _tc_body(x_ref, coef_ref, w1_ref, b1_ref, w2_ref, b2_ref, o_ref):
    x = x_ref[...].astype(jnp.bfloat16)
    coef = coef_ref[...]
    w1 = w1_ref[...].astype(jnp.bfloat16)
    pre = jnp.dot(coef, b1_ref[...], preferred_element_type=jnp.float32)
    for k in range(_K):
        pk = jnp.dot(x, w1[k], preferred_element_type=jnp.float32)
        pre = pre + (pk if k == 0 else coef[:, k:k + 1] * pk)
    h = jnp.maximum(pre, 0.0).astype(jnp.bfloat16)
    w2 = w2_ref[...].astype(jnp.bfloat16)
    out = jnp.dot(coef, b2_ref[...], preferred_element_type=jnp.float32)
    for k in range(_K):
        qk = jnp.dot(h, w2[k], preferred_element_type=jnp.float32)
        out = out + (qk if k == 0 else coef[:, k:k + 1] * qk)
    o_ref[...] = out


_TC_GRID = (_B // _BLK,)
_TC_IN_SPECS = [
    pl.BlockSpec((_BLK, _D_IN), lambda i: (i, 0)),
    pl.BlockSpec((_BLK, _KP), lambda i: (i, 0)),
    pl.BlockSpec((_K, _D_IN, _D_H), lambda i: (0, 0, 0)),
    pl.BlockSpec((_KP, _D_H), lambda i: (0, 0)),
    pl.BlockSpec((_K, _D_H, _D_OUT), lambda i: (0, 0, 0)),
    pl.BlockSpec((_KP, _D_OUT), lambda i: (0, 0)),
]
_TC_OUT_SPEC = pl.BlockSpec((_BLK, _D_OUT), lambda i: (i, 0))


def _sc_gather_coef(table, ids_2d):
    mesh = plsc.VectorSubcoreMesh(core_axis_name="core",
                                  subcore_axis_name="subcore")

    @pl.kernel(out_type=jax.ShapeDtypeStruct((_B, _KP), jnp.float32),
               mesh=mesh)
    def gather_kernel(tab_hbm, ids_hbm, o_hbm):
        def body(i_vmem, o_vmem):
            pltpu.sync_copy(tab_hbm.at[i_vmem.at[0]], o_vmem)

        pltpu.emit_pipeline(
            body,
            grid=(_B // _GW,),
            in_specs=[pl.BlockSpec((1, _GW), lambda i: (0, i))],
            out_specs=[pl.BlockSpec((_GW, _KP), lambda i: (i, 0))],
            core_axis_name=("core", "subcore"),
            dimension_semantics=(pltpu.PARALLEL,),
        )(ids_hbm, o_hbm)

    return gather_kernel(table, ids_2d)


def kernel(x, xtype_ids, mesa_layer_weight, meta_layer_weight,
           meta_layer_bias, base_state):
    base = base_state + meta_layer_bias
    wallT = jnp.concatenate([base[None, :], meta_layer_weight.T], axis=0)
    b1p = jnp.zeros((_KP, _D_H), jnp.float32).at[:_K].set(wallT[:, _S1:_S2])
    w1p = wallT[:, :_S1].reshape(_K, _D_IN, _D_H)
    w2p = wallT[:, _S2:_S3].reshape(_K, _D_H, _D_OUT)
    b2p = jnp.zeros((_KP, _D_OUT), jnp.float32).at[:_K].set(wallT[:, _S3:])
    tab = (jnp.zeros((_N_TYPES, _KP), jnp.float32)
           .at[:, 0].set(1.0)
           .at[:, 1:1 + _MESA].set(mesa_layer_weight.T))

    coef = _sc_gather_coef(tab, xtype_ids.reshape(1, _B).astype(jnp.int32))

    return pl.pallas_call(
        _tc_body,
        grid=_TC_GRID,
        in_specs=_TC_IN_SPECS,
        out_specs=_TC_OUT_SPEC,
        out_shape=jax.ShapeDtypeStruct((_B, _D_OUT), jnp.float32),
        compiler_params=pltpu.CompilerParams(
            dimension_semantics=("parallel",)),
    )(x, coef, w1p, b1p, w2p, b2p)


# speedup vs baseline: 1.0216x; 1.0216x over previous
import jax
import jax.numpy as jnp
from jax.experimental import pallas as pl
from jax.experimental.pallas import tpu as pltpu
from jax.experimental.pallas import tpu_sc as plsc

_B = 4096
_D_IN = 128
_D_H = 256
_D_OUT = 64
_N_TYPES = 256
_MESA = 8
_S1 = _D_IN * _D_H
_S2 = _S1 + _D_H
_S3 = _S2 + _D_H * _D_OUT
_K = _MESA + 1
_KP = 128

_BLK = 1024
_GW = 128


def _tc_body(x_ref, coef_ref, w1_ref, b1_ref, w2_ref, b2_ref, o_ref):
    x = x_ref[...].astype(jnp.bfloat16)
    coef = coef_ref[...]
    w1 = w1_ref[...].astype(jnp.bfloat16)
    pre = jnp.dot(coef, b1_ref[...], preferred_element_type=jnp.float32)
    for k in range(_K):
        pk = jnp.dot(x, w1[k], preferred_element_type=jnp.float32)
        pre = pre + (pk if k == 0 else coef[:, k:k + 1] * pk)
    h = jnp.maximum(pre, 0.0).astype(jnp.bfloat16)
    w2 = w2_ref[...].astype(jnp.bfloat16)
    out = jnp.dot(coef, b2_ref[...], preferred_element_type=jnp.float32)
    for k in range(_K):
        qk = jnp.dot(h, w2[k], preferred_element_type=jnp.float32)
        out = out + (qk if k == 0 else coef[:, k:k + 1] * qk)
    o_ref[...] = out


_TC_GRID = (_B // _BLK,)
_TC_IN_SPECS = [
    pl.BlockSpec((_BLK, _D_IN), lambda i: (i, 0)),
    pl.BlockSpec((_BLK, _KP), lambda i: (i, 0)),
    pl.BlockSpec((_K, _D_IN, _D_H), lambda i: (0, 0, 0)),
    pl.BlockSpec((_KP, _D_H), lambda i: (0, 0)),
    pl.BlockSpec((_K, _D_H, _D_OUT), lambda i: (0, 0, 0)),
    pl.BlockSpec((_KP, _D_OUT), lambda i: (0, 0)),
]
_TC_OUT_SPEC = pl.BlockSpec((_BLK, _D_OUT), lambda i: (i, 0))


def _sc_gather_coef(table, ids_2d):
    mesh = plsc.VectorSubcoreMesh(core_axis_name="core",
                                  subcore_axis_name="subcore")

    @pl.kernel(out_type=jax.ShapeDtypeStruct((_B, _KP), jnp.float32),
               mesh=mesh)
    def gather_kernel(tab_hbm, ids_hbm, o_hbm):
        def body(i_vmem, o_vmem):
            pltpu.sync_copy(tab_hbm.at[i_vmem.at[0]], o_vmem)

        pltpu.emit_pipeline(
            body,
            grid=(_B // _GW,),
            in_specs=[pl.BlockSpec((1, _GW), lambda i: (0, i))],
            out_specs=[pl.BlockSpec((_GW, _KP), lambda i: (i, 0))],
            core_axis_name=("core", "subcore"),
            dimension_semantics=(pltpu.PARALLEL,),
        )(ids_hbm, o_hbm)

    return gather_kernel(table, ids_2d)


def kernel(x, xtype_ids, mesa_layer_weight, meta_layer_weight,
           meta_layer_bias, base_state):
    base = base_state + meta_layer_bias
    wallT = jnp.concatenate([base[None, :], meta_layer_weight.T], axis=0)
    b1p = jnp.zeros((_KP, _D_H), jnp.float32).at[:_K].set(wallT[:, _S1:_S2])
    w1p = wallT[:, :_S1].reshape(_K, _D_IN, _D_H)
    w2p = wallT[:, _S2:_S3].reshape(_K, _D_H, _D_OUT)
    b2p = jnp.zeros((_KP, _D_OUT), jnp.float32).at[:_K].set(wallT[:, _S3:])
    tab = (jnp.zeros((_N_TYPES, _KP), jnp.float32)
           .at[:, 0].set(1.0)
           .at[:, 1:1 + _MESA].set(mesa_layer_weight.T))

    coef = _sc_gather_coef(tab, xtype_ids.reshape(1, _B).astype(jnp.int32))

    return pl.pallas_call(
        _tc_body,
        grid=_TC_GRID,
        in_specs=_TC_IN_SPECS,
        out_specs=_TC_OUT_SPEC,
        out_shape=jax.ShapeDtypeStruct((_B, _D_OUT), jnp.float32),
    )(x, coef, w1p, b1p, w2p, b2p)
